# Initial kernel scaffold; baseline (speedup 1.0000x reference)
import jax
import jax.numpy as jnp
from jax.experimental import pallas as pl


def kernel(x, W):
    raise NotImplementedError("write your pallas kernel here")


# speedup vs baseline: 1.1795x; 1.1795x over previous
import functools

import jax
import jax.numpy as jnp
from jax import lax
from jax.experimental import pallas as pl
from jax.experimental.pallas import tpu as pltpu
from jax.experimental.pallas import tpu_sc as plsc

_K = 8192
_C = 256
_N = 8192

_N_BLK = 512
_K_BLK = 2048

_NUM_WORKERS = 32
_B_PER_W = _N // _NUM_WORKERS


def _argmin_body(a_ref, b_ref, x_ref, w_ref, idx_ref, minv_ref,
                 run_min, run_idx, run_sel):
    k = pl.program_id(1)
    nk = pl.num_programs(1)

    m2 = lax.dot_general(
        x_ref[...], w_ref[...], (((1,), (1,)), ((), ())),
        preferred_element_type=jnp.float32)
    s = (a_ref[...] + b_ref[...]) - 2.0 * m2

    mcol = jnp.min(s, axis=1, keepdims=True)
    kio = lax.broadcasted_iota(jnp.int32, s.shape, 1)
    amin = jnp.min(jnp.where(s == mcol, kio, _K_BLK), axis=1, keepdims=True)
    gidx = amin + k * _K_BLK
    mbf = mcol.astype(jnp.bfloat16).astype(jnp.float32)

    @pl.when(k == 0)
    def _():
        run_min[...] = mbf
        run_idx[...] = gidx
        run_sel[...] = mcol

    @pl.when(k > 0)
    def _():
        better = mcol < run_min[...]
        run_idx[...] = jnp.where(better, gidx, run_idx[...])
        run_sel[...] = jnp.where(better, mcol, run_sel[...])
        run_min[...] = jnp.where(better, mbf, run_min[...])

    @pl.when(k == nk - 1)
    def _():
        idx_ref[...] = run_idx[...]
        minv_ref[...] = run_sel[...]


def _argmin_call(flat, a, b, w):
    grid = (_N // _N_BLK, _K // _K_BLK)
    return pl.pallas_call(
        _argmin_body,
        grid=grid,
        in_specs=[
            pl.BlockSpec((_N_BLK, 1), lambda n, k: (n, 0)),
            pl.BlockSpec((1, _K_BLK), lambda n, k: (0, k)),
            pl.BlockSpec((_N_BLK, _C), lambda n, k: (n, 0)),
            pl.BlockSpec((_K_BLK, _C), lambda n, k: (k, 0)),
        ],
        out_specs=[
            pl.BlockSpec((_N_BLK, 1), lambda n, k: (n, 0)),
            pl.BlockSpec((_N_BLK, 1), lambda n, k: (n, 0)),
        ],
        out_shape=[
            jax.ShapeDtypeStruct((_N, 1), jnp.int32),
            jax.ShapeDtypeStruct((_N, 1), jnp.float32),
        ],
        scratch_shapes=[
            pltpu.VMEM((_N_BLK, 1), jnp.float32),
            pltpu.VMEM((_N_BLK, 1), jnp.int32),
            pltpu.VMEM((_N_BLK, 1), jnp.float32),
        ],
        compiler_params=pltpu.CompilerParams(
            dimension_semantics=("arbitrary", "arbitrary")),
    )(a, b, flat, w)


def _gather_body(table_hbm, idx_hbm, out_hbm, idx_v, rows_v, sem):
    wid = lax.axis_index("s") * 2 + lax.axis_index("c")
    base = wid * _B_PER_W
    pltpu.sync_copy(idx_hbm.at[pl.ds(base, _B_PER_W)], idx_v)
    pltpu.async_copy(table_hbm.at[idx_v], rows_v, sem).wait()
    pltpu.sync_copy(rows_v, out_hbm.at[pl.ds(base, _B_PER_W)])


@functools.cache
def _gather_call():
    return pl.kernel(
        _gather_body,
        mesh=plsc.VectorSubcoreMesh(core_axis_name="c", subcore_axis_name="s"),
        out_type=jax.ShapeDtypeStruct((_N, _C), jnp.float32),
        scratch_types=[
            pltpu.VMEM((_B_PER_W,), jnp.int32),
            pltpu.VMEM((_B_PER_W, _C), jnp.float32),
            pltpu.SemaphoreType.DMA,
        ],
    )


def kernel(x, W):
    xf = jnp.transpose(x, (0, 2, 3, 4, 1))
    flat = xf.reshape(-1, _C)
    a = jnp.sum(flat ** 2, axis=1, keepdims=True)
    b = jnp.sum(W ** 2, axis=1)[None, :]

    idx2d, minv = _argmin_call(flat, a, b, W)
    idx = idx2d[:, 0]

    q = _gather_call()(W, idx)

    m = jnp.sum(minv) / jnp.float32(_N * _C)
    loss = m + jnp.float32(0.25) * m

    quantized = q.reshape(xf.shape)
    quantized = jnp.transpose(quantized, (0, 4, 1, 2, 3))
    return quantized, loss


# speedup vs baseline: 1.2984x; 1.1008x over previous
import functools

import jax
import jax.numpy as jnp
from jax import lax
from jax.experimental import pallas as pl
from jax.experimental.pallas import tpu as pltpu
from jax.experimental.pallas import tpu_sc as plsc

_K = 8192
_C = 256
_N = 8192

_N_BLK = 1024
_K_BLK = 2048

_NUM_WORKERS = 32
_B_PER_W = _N // _NUM_WORKERS


def _argmin_body(a_ref, b_ref, x_ref, w_ref, idx_ref, minv_ref,
                 run_min, run_idx, run_sel):
    k = pl.program_id(1)
    nk = pl.num_programs(1)

    m2 = lax.dot_general(
        x_ref[...], w_ref[...], (((1,), (1,)), ((), ())),
        preferred_element_type=jnp.float32)
    s = (a_ref[...] + b_ref[...]) - 2.0 * m2

    mcol = jnp.min(s, axis=1, keepdims=True)
    kio = lax.broadcasted_iota(jnp.int32, s.shape, 1)
    amin = jnp.min(jnp.where(s == mcol, kio, _K_BLK), axis=1, keepdims=True)
    gidx = amin + k * _K_BLK
    mbf = mcol.astype(jnp.bfloat16).astype(jnp.float32)

    @pl.when(k == 0)
    def _():
        run_min[...] = mbf
        run_idx[...] = gidx
        run_sel[...] = mcol

    @pl.when(k > 0)
    def _():
        better = mcol < run_min[...]
        run_idx[...] = jnp.where(better, gidx, run_idx[...])
        run_sel[...] = jnp.where(better, mcol, run_sel[...])
        run_min[...] = jnp.where(better, mbf, run_min[...])

    @pl.when(k == nk - 1)
    def _():
        idx_ref[...] = run_idx[...]
        minv_ref[...] = run_sel[...]


def _argmin_call(flat, a, b, w):
    grid = (_N // _N_BLK, _K // _K_BLK)
    return pl.pallas_call(
        _argmin_body,
        grid=grid,
        in_specs=[
            pl.BlockSpec((_N_BLK, 1), lambda n, k: (n, 0)),
            pl.BlockSpec((1, _K_BLK), lambda n, k: (0, k)),
            pl.BlockSpec((_N_BLK, _C), lambda n, k: (n, 0)),
            pl.BlockSpec((_K_BLK, _C), lambda n, k: (k, 0)),
        ],
        out_specs=[
            pl.BlockSpec((_N_BLK, 1), lambda n, k: (n, 0)),
            pl.BlockSpec((_N_BLK, 1), lambda n, k: (n, 0)),
        ],
        out_shape=[
            jax.ShapeDtypeStruct((_N, 1), jnp.int32),
            jax.ShapeDtypeStruct((_N, 1), jnp.float32),
        ],
        scratch_shapes=[
            pltpu.VMEM((_N_BLK, 1), jnp.float32),
            pltpu.VMEM((_N_BLK, 1), jnp.int32),
            pltpu.VMEM((_N_BLK, 1), jnp.float32),
        ],
        compiler_params=pltpu.CompilerParams(
            dimension_semantics=("arbitrary", "arbitrary")),
    )(a, b, flat, w)


def _gather_body(table_hbm, idx_hbm, out_hbm, idx_v, rows_v, sem):
    wid = lax.axis_index("s") * 2 + lax.axis_index("c")
    base = wid * _B_PER_W
    pltpu.sync_copy(idx_hbm.at[pl.ds(base, _B_PER_W)], idx_v)
    pltpu.async_copy(table_hbm.at[idx_v], rows_v, sem).wait()
    pltpu.sync_copy(rows_v, out_hbm.at[pl.ds(base, _B_PER_W)])


@functools.cache
def _gather_call():
    return pl.kernel(
        _gather_body,
        mesh=plsc.VectorSubcoreMesh(core_axis_name="c", subcore_axis_name="s"),
        out_type=jax.ShapeDtypeStruct((_N, _C), jnp.float32),
        scratch_types=[
            pltpu.VMEM((_B_PER_W,), jnp.int32),
            pltpu.VMEM((_B_PER_W, _C), jnp.float32),
            pltpu.SemaphoreType.DMA,
        ],
    )


def kernel(x, W):
    xf = jnp.transpose(x, (0, 2, 3, 4, 1))
    flat = xf.reshape(-1, _C)
    a = jnp.sum(flat ** 2, axis=1, keepdims=True)
    b = jnp.sum(W ** 2, axis=1)[None, :]

    idx2d, minv = _argmin_call(flat, a, b, W)
    idx = idx2d[:, 0]

    q = _gather_call()(W, idx)

    m = jnp.sum(minv) / jnp.float32(_N * _C)
    loss = m + jnp.float32(0.25) * m

    quantized = q.reshape(xf.shape)
    quantized = jnp.transpose(quantized, (0, 4, 1, 2, 3))
    return quantized, loss
